# baseline (speedup 1.0000x reference)
import functools

import jax
import jax.numpy as jnp
from jax import lax
from jax.experimental import pallas as pl
from jax.experimental.pallas import tpu as pltpu
from jax.experimental.pallas import tpu_sc as plsc

EMB = 32
B_ROWS = 4096
B_COLS = 200
NUM_WORKERS = 32
R_PER_W = B_ROWS // NUM_WORKERS


def _emb_body(idx_hbm, table_hbm, out_hbm, idx_v, rows_v, gsem0, gsem1, ssem0, ssem1):
    nc = 2
    wid = lax.axis_index("s") * nc + lax.axis_index("c")
    base_row = wid * R_PER_W
    gsems = (gsem0, gsem1)
    ssems = (ssem0, ssem1)

    pltpu.sync_copy(idx_hbm.at[pl.ds(base_row, R_PER_W)], idx_v)

    def fire_gather(j, p):
        pltpu.async_copy(table_hbm.at[idx_v.at[j]], rows_v.at[p], gsems[p])

    def wait_gather(p):
        pltpu.make_async_copy(table_hbm.at[idx_v.at[0]], rows_v.at[p], gsems[p]).wait()

    def fire_store(j, p):
        pltpu.async_copy(rows_v.at[p], out_hbm.at[base_row + j], ssems[p])

    def wait_store(p):
        pltpu.make_async_copy(rows_v.at[p], out_hbm.at[base_row], ssems[p]).wait()

    fire_gather(0, 0)

    def group(g, carry):
        for p in (0, 1):
            j = 2 * g + p
            q = 1 - p

            @pl.when(j >= 1)
            def _():
                wait_store(q)

            @pl.when(j + 1 < R_PER_W)
            def _():
                fire_gather(j + 1, q)

            wait_gather(p)
            fire_store(j, p)
        return carry

    lax.fori_loop(0, R_PER_W // 2, group, 0)
    wait_store((R_PER_W - 1) % 2)


@functools.partial(
    pl.kernel,
    mesh=plsc.VectorSubcoreMesh(core_axis_name="c", subcore_axis_name="s"),
    out_type=jax.ShapeDtypeStruct((B_ROWS, B_COLS, EMB), jnp.float32),
    scratch_types=[
        pltpu.VMEM((R_PER_W, B_COLS), jnp.int32),
        pltpu.VMEM((2, B_COLS, EMB), jnp.float32),
        pltpu.SemaphoreType.DMA,
        pltpu.SemaphoreType.DMA,
        pltpu.SemaphoreType.DMA,
        pltpu.SemaphoreType.DMA,
    ],
    compiler_params=pltpu.CompilerParams(use_tc_tiling_on_sc=False),
)
def _emb_lookup(idx_hbm, table_hbm, out_hbm, idx_v, rows_v, gsem0, gsem1, ssem0, ssem1):
    _emb_body(idx_hbm, table_hbm, out_hbm, idx_v, rows_v, gsem0, gsem1, ssem0, ssem1)


def kernel(vocab_ids, node_embs_weight):
    return _emb_lookup(vocab_ids.astype(jnp.int32), node_embs_weight)
